# baseline (speedup 1.0000x reference)
import functools

import jax
import jax.numpy as jnp
from jax import lax
from jax.experimental import pallas as pl
from jax.experimental.pallas import tpu as pltpu
from jax.experimental.pallas import tpu_sc as plsc

_B = 16384
_U = 1000000
_CHU = 16384
_GRID_PACK = (_U + _CHU - 1) // _CHU
_RP = _GRID_PACK * (_CHU // 4)
_UPAD = _RP * 4
_NC = 2
_NS = 16
_NW = _NC * _NS
_BPW = _B // _NW
_CH = 128
_NCH = _BPW // _CH
_FG = 32
_FM = 128


def _pack_body(xu_ref, xi_ref, ou_ref, oi_ref):
    q_span = _CHU // 4
    eyes = [jnp.eye(32, 128, k=32 * q, dtype=jnp.float32) for q in range(4)]
    for x_ref, o_ref in ((xu_ref, ou_ref), (xi_ref, oi_ref)):
        acc = None
        for q in range(4):
            t = jax.lax.dot_general(
                x_ref[:, pl.ds(q * q_span, q_span)], eyes[q],
                (((0,), (0,)), ((), ())),
                preferred_element_type=jnp.float32)
            acc = t if acc is None else acc + t
        o_ref[...] = acc


def _pack_tables(eug, eig):
    pu, pi = pl.pallas_call(
        _pack_body,
        grid=(_GRID_PACK,),
        in_specs=[pl.BlockSpec((32, _CHU), lambda j: (0, j)),
                  pl.BlockSpec((32, _CHU), lambda j: (0, j))],
        out_specs=[pl.BlockSpec((_CHU // 4, 128), lambda j: (j, 0)),
                   pl.BlockSpec((_CHU // 4, 128), lambda j: (j, 0))],
        out_shape=[jax.ShapeDtypeStruct((_RP, 128), jnp.float32),
                   jax.ShapeDtypeStruct((_RP, 128), jnp.float32)],
    )(eug.T, eig.T)
    return pu.reshape(_UPAD, 32), pi.reshape(_UPAD, 32)


def _mesh():
    return plsc.VectorSubcoreMesh(core_axis_name="c", subcore_axis_name="s",
                                  num_cores=_NC, num_subcores=_NS)


def _wid():
    return lax.axis_index("s") * _NC + lax.axis_index("c")


def _gmf_body(user_hbm, item_hbm, eug, eig, gmf_out,
              idx_u, idx_i, vid_u, vid_i, gu_v, gi_v, sem):
    wid = _wid()
    pltpu.sync_copy(user_hbm.at[wid], idx_u)
    pltpu.sync_copy(item_hbm.at[wid], idx_i)

    sb = _CHU.bit_length() - 1
    ss = sb - 2
    sm = (1 << ss) - 1

    def remap(k, carry):
        for src, dst in ((idx_u, vid_u), (idx_i, vid_i)):
            for c in range(_NCH):
                v = src[c, pl.ds(k * 16, 16)]
                vrow = (jax.lax.shift_left(jax.lax.shift_right_logical(v, sb),
                                           sb)
                        + jax.lax.shift_left(v & sm, 2)
                        + (jax.lax.shift_right_logical(v, ss) & 3))
                dst[c, pl.ds(k * 16, 16)] = vrow
        return carry

    lax.fori_loop(0, _CH // 16, remap, 0)
    descs = []
    for c in range(_NCH):
        descs.append(pltpu.async_copy(eug.at[vid_u.at[c]], gu_v.at[c], sem))
        descs.append(pltpu.async_copy(eig.at[vid_i.at[c]], gi_v.at[c], sem))
    for d in descs:
        d.wait()

    def mul_row(r, carry):
        for c in range(_NCH):
            for j in range(_FG // 16):
                sl = pl.ds(j * 16, 16)
                gu_v[c, r, sl] = gu_v[c, r, sl] * gi_v[c, r, sl]
        return carry

    lax.fori_loop(0, _CH, mul_row, 0)
    pltpu.sync_copy(gu_v, gmf_out.at[wid])


def _sc_gmf(user3, item3, eug, eig):
    k = functools.partial(
        pl.kernel, mesh=_mesh(),
        compiler_params=pltpu.CompilerParams(use_tc_tiling_on_sc=False),
        out_type=jax.ShapeDtypeStruct((_NW, _NCH, _CH, _FG), jnp.float32),
        scratch_types=[
            pltpu.VMEM((_NCH, _CH), jnp.int32),
            pltpu.VMEM((_NCH, _CH), jnp.int32),
            pltpu.VMEM((_NCH, _CH), jnp.int32),
            pltpu.VMEM((_NCH, _CH), jnp.int32),
            pltpu.VMEM((_NCH, _CH, _FG), jnp.float32),
            pltpu.VMEM((_NCH, _CH, _FG), jnp.float32),
            pltpu.SemaphoreType.DMA,
        ],
    )(_gmf_body)
    return k(user3, item3, eug, eig).reshape(_B, _FG)


def _mlp_gather_body(user_hbm, item_hbm, eum, eim, mu_out, mi_out,
                     idx_u, idx_i, bufs_u, bufs_i, sem_u, sem_i):
    wid = _wid()
    base = wid * _BPW
    for c in range(_NCH):
        pltpu.sync_copy(user_hbm.at[pl.ds(base + c * _CH, _CH)], idx_u.at[c])
        pltpu.sync_copy(item_hbm.at[pl.ds(base + c * _CH, _CH)], idx_i.at[c])
    du = [None, None]
    di = [None, None]
    du[0] = pltpu.async_copy(eum.at[idx_u.at[0]], bufs_u.at[0], sem_u.at[0])
    di[0] = pltpu.async_copy(eim.at[idx_i.at[0]], bufs_i.at[0], sem_i.at[0])
    for c in range(_NCH):
        s = c % 2
        if c + 1 < _NCH:
            n = (c + 1) % 2
            du[n] = pltpu.async_copy(eum.at[idx_u.at[c + 1]], bufs_u.at[n],
                                     sem_u.at[n])
            di[n] = pltpu.async_copy(eim.at[idx_i.at[c + 1]], bufs_i.at[n],
                                     sem_i.at[n])
        du[s].wait()
        pltpu.sync_copy(bufs_u.at[s], mu_out.at[pl.ds(base + c * _CH, _CH)])
        di[s].wait()
        pltpu.sync_copy(bufs_i.at[s], mi_out.at[pl.ds(base + c * _CH, _CH)])


def _sc_mlp_gather(user, item, eum, eim):
    k = functools.partial(
        pl.kernel, mesh=_mesh(),
        out_type=(
            jax.ShapeDtypeStruct((_B, _FM), jnp.float32),
            jax.ShapeDtypeStruct((_B, _FM), jnp.float32),
        ),
        scratch_types=[
            pltpu.VMEM((_NCH, _CH), jnp.int32),
            pltpu.VMEM((_NCH, _CH), jnp.int32),
            pltpu.VMEM((2, _CH, _FM), jnp.float32),
            pltpu.VMEM((2, _CH, _FM), jnp.float32),
            pltpu.SemaphoreType.DMA((2,)),
            pltpu.SemaphoreType.DMA((2,)),
        ],
    )(_mlp_gather_body)
    return k(user, item, eum, eim)


_BLK = 2048


def _mlp_body(mu_ref, mi_ref, gmf_ref, w1a, w1b, b1, w2, b2, w3, b3,
              wpa, wpb, bp, out_ref):
    x = jnp.dot(mu_ref[...], w1a[...], preferred_element_type=jnp.float32)
    x = x + jnp.dot(mi_ref[...], w1b[...], preferred_element_type=jnp.float32)
    x = jnp.maximum(x + b1[...], 0.0)
    x = jnp.maximum(jnp.dot(x, w2[...], preferred_element_type=jnp.float32)
                    + b2[...], 0.0)
    x = jnp.maximum(jnp.dot(x, w3[...], preferred_element_type=jnp.float32)
                    + b3[...], 0.0)
    out = jnp.dot(gmf_ref[...], wpa[...], preferred_element_type=jnp.float32)
    out = out + jnp.dot(x, wpb[...], preferred_element_type=jnp.float32)
    out_ref[...] = out + bp[...]


def _tc_mlp(mu, mi, gmf, W1, b1, W2, b2, W3, b3, Wp, bp):
    full = lambda shape: pl.BlockSpec(shape, lambda i: (0, 0))
    grid = (_B // _BLK,)
    return pl.pallas_call(
        _mlp_body,
        grid=grid,
        in_specs=[
            pl.BlockSpec((_BLK, _FM), lambda i: (i, 0)),
            pl.BlockSpec((_BLK, _FM), lambda i: (i, 0)),
            pl.BlockSpec((_BLK, _FG), lambda i: (i, 0)),
            full((128, 128)), full((128, 128)), full((1, 128)),
            full((128, 64)), full((1, 64)),
            full((64, 32)), full((1, 32)),
            full((32, 1)), full((32, 1)), full((1, 1)),
        ],
        out_specs=pl.BlockSpec((_BLK, 1), lambda i: (i, 0)),
        out_shape=jax.ShapeDtypeStruct((_B, 1), jnp.float32),
    )(mu, mi, gmf,
      W1[:128], W1[128:], b1.reshape(1, -1),
      W2, b2.reshape(1, -1), W3, b3.reshape(1, -1),
      Wp[:32], Wp[32:], bp.reshape(1, 1))


def kernel(user, item, embed_user_GMF, embed_item_GMF, embed_user_MLP,
           embed_item_MLP, W1, b1, W2, b2, W3, b3, Wp, bp):
    user = user.astype(jnp.int32)
    item = item.astype(jnp.int32)
    mu, mi = _sc_mlp_gather(user, item, embed_user_MLP, embed_item_MLP)
    pug, pig = _pack_tables(embed_user_GMF, embed_item_GMF)
    gmf = _sc_gmf(user.reshape(_NW, _NCH, _CH), item.reshape(_NW, _NCH, _CH),
                  pug, pig)
    out = _tc_mlp(mu, mi, gmf, W1, b1, W2, b2, W3, b3, Wp, bp)
    return out.reshape(-1)


# speedup vs baseline: 1.5142x; 1.5142x over previous
import functools

import jax
import jax.numpy as jnp
from jax import lax
from jax.experimental import pallas as pl
from jax.experimental.pallas import tpu as pltpu
from jax.experimental.pallas import tpu_sc as plsc

_B = 16384
_U = 1000000
_CHU = 16384
_GRID_PACK = (_U + _CHU - 1) // _CHU
_RP = _GRID_PACK * (_CHU // 4)
_UPAD = _RP * 4
_NC = 2
_NS = 16
_NW = _NC * _NS
_BPW = _B // _NW
_CH = 128
_NCH = _BPW // _CH
_FG = 32
_FM = 128


def _pack_body(xu_ref, xi_ref, ou_ref, oi_ref):
    q_span = _CHU // 4
    eye = jnp.eye(128, dtype=jnp.float32)
    j = pl.program_id(0)
    valid = _U - j * _CHU
    col = lax.broadcasted_iota(jnp.int32, (32, _CHU), 1)
    in_bounds = col < valid
    for x_ref, o_ref in ((xu_ref, ou_ref), (xi_ref, oi_ref)):
        x = jnp.where(in_bounds, x_ref[...], 0.0)
        xc = jnp.concatenate([x[:, q * q_span:(q + 1) * q_span]
                              for q in range(4)], axis=0)
        o_ref[...] = jax.lax.dot_general(
            xc, eye, (((0,), (0,)), ((), ())),
            preferred_element_type=jnp.float32)


def _pack_tables(eug, eig):
    pu, pi = pl.pallas_call(
        _pack_body,
        grid=(_GRID_PACK,),
        in_specs=[pl.BlockSpec((32, _CHU), lambda j: (0, j)),
                  pl.BlockSpec((32, _CHU), lambda j: (0, j))],
        out_specs=[pl.BlockSpec((_CHU // 4, 128), lambda j: (j, 0)),
                   pl.BlockSpec((_CHU // 4, 128), lambda j: (j, 0))],
        out_shape=[jax.ShapeDtypeStruct((_RP, 128), jnp.float32),
                   jax.ShapeDtypeStruct((_RP, 128), jnp.float32)],
    )(eug.T, eig.T)
    return pu.reshape(_UPAD, 32), pi.reshape(_UPAD, 32)


def _mesh():
    return plsc.VectorSubcoreMesh(core_axis_name="c", subcore_axis_name="s",
                                  num_cores=_NC, num_subcores=_NS)


def _wid():
    return lax.axis_index("s") * _NC + lax.axis_index("c")


def _gmf_body(user_hbm, item_hbm, eug, eig, gmf_out,
              idx_u, idx_i, vid_u, vid_i, gu_v, gi_v, sem):
    wid = _wid()
    pltpu.sync_copy(user_hbm.at[wid], idx_u)
    pltpu.sync_copy(item_hbm.at[wid], idx_i)

    sb = _CHU.bit_length() - 1
    ss = sb - 2
    sm = (1 << ss) - 1

    def remap(k, carry):
        for src, dst in ((idx_u, vid_u), (idx_i, vid_i)):
            for c in range(_NCH):
                v = src[c, pl.ds(k * 16, 16)]
                vrow = (jax.lax.shift_left(jax.lax.shift_right_logical(v, sb),
                                           sb)
                        + jax.lax.shift_left(v & sm, 2)
                        + (jax.lax.shift_right_logical(v, ss) & 3))
                dst[c, pl.ds(k * 16, 16)] = vrow
        return carry

    lax.fori_loop(0, _CH // 16, remap, 0)
    descs = []
    for c in range(_NCH):
        descs.append(pltpu.async_copy(eug.at[vid_u.at[c]], gu_v.at[c], sem))
        descs.append(pltpu.async_copy(eig.at[vid_i.at[c]], gi_v.at[c], sem))
    for d in descs:
        d.wait()

    def mul_row(r, carry):
        for c in range(_NCH):
            for j in range(_FG // 16):
                sl = pl.ds(j * 16, 16)
                gu_v[c, r, sl] = gu_v[c, r, sl] * gi_v[c, r, sl]
        return carry

    lax.fori_loop(0, _CH, mul_row, 0)
    pltpu.sync_copy(gu_v, gmf_out.at[wid])


def _sc_gmf(user3, item3, eug, eig):
    k = functools.partial(
        pl.kernel, mesh=_mesh(),
        compiler_params=pltpu.CompilerParams(use_tc_tiling_on_sc=False),
        out_type=jax.ShapeDtypeStruct((_NW, _NCH, _CH, _FG), jnp.float32),
        scratch_types=[
            pltpu.VMEM((_NCH, _CH), jnp.int32),
            pltpu.VMEM((_NCH, _CH), jnp.int32),
            pltpu.VMEM((_NCH, _CH), jnp.int32),
            pltpu.VMEM((_NCH, _CH), jnp.int32),
            pltpu.VMEM((_NCH, _CH, _FG), jnp.float32),
            pltpu.VMEM((_NCH, _CH, _FG), jnp.float32),
            pltpu.SemaphoreType.DMA,
        ],
    )(_gmf_body)
    return k(user3, item3, eug, eig).reshape(_B, _FG)


def _mlp_gather_body(user_hbm, item_hbm, eum, eim, mu_out, mi_out,
                     idx_u, idx_i, bufs_u, bufs_i, sem_u, sem_i):
    wid = _wid()
    base = wid * _BPW
    for c in range(_NCH):
        pltpu.sync_copy(user_hbm.at[pl.ds(base + c * _CH, _CH)], idx_u.at[c])
        pltpu.sync_copy(item_hbm.at[pl.ds(base + c * _CH, _CH)], idx_i.at[c])
    du = [None, None]
    di = [None, None]
    du[0] = pltpu.async_copy(eum.at[idx_u.at[0]], bufs_u.at[0], sem_u.at[0])
    di[0] = pltpu.async_copy(eim.at[idx_i.at[0]], bufs_i.at[0], sem_i.at[0])
    for c in range(_NCH):
        s = c % 2
        if c + 1 < _NCH:
            n = (c + 1) % 2
            du[n] = pltpu.async_copy(eum.at[idx_u.at[c + 1]], bufs_u.at[n],
                                     sem_u.at[n])
            di[n] = pltpu.async_copy(eim.at[idx_i.at[c + 1]], bufs_i.at[n],
                                     sem_i.at[n])
        du[s].wait()
        pltpu.sync_copy(bufs_u.at[s], mu_out.at[pl.ds(base + c * _CH, _CH)])
        di[s].wait()
        pltpu.sync_copy(bufs_i.at[s], mi_out.at[pl.ds(base + c * _CH, _CH)])


def _sc_mlp_gather(user, item, eum, eim):
    k = functools.partial(
        pl.kernel, mesh=_mesh(),
        out_type=(
            jax.ShapeDtypeStruct((_B, _FM), jnp.float32),
            jax.ShapeDtypeStruct((_B, _FM), jnp.float32),
        ),
        scratch_types=[
            pltpu.VMEM((_NCH, _CH), jnp.int32),
            pltpu.VMEM((_NCH, _CH), jnp.int32),
            pltpu.VMEM((2, _CH, _FM), jnp.float32),
            pltpu.VMEM((2, _CH, _FM), jnp.float32),
            pltpu.SemaphoreType.DMA((2,)),
            pltpu.SemaphoreType.DMA((2,)),
        ],
    )(_mlp_gather_body)
    return k(user, item, eum, eim)


_BLK = 2048


def _mlp_body(mu_ref, mi_ref, gmf_ref, w1a, w1b, b1, w2, b2, w3, b3,
              wpa, wpb, bp, out_ref):
    x = jnp.dot(mu_ref[...], w1a[...], preferred_element_type=jnp.float32)
    x = x + jnp.dot(mi_ref[...], w1b[...], preferred_element_type=jnp.float32)
    x = jnp.maximum(x + b1[...], 0.0)
    x = jnp.maximum(jnp.dot(x, w2[...], preferred_element_type=jnp.float32)
                    + b2[...], 0.0)
    x = jnp.maximum(jnp.dot(x, w3[...], preferred_element_type=jnp.float32)
                    + b3[...], 0.0)
    out = jnp.dot(gmf_ref[...], wpa[...], preferred_element_type=jnp.float32)
    out = out + jnp.dot(x, wpb[...], preferred_element_type=jnp.float32)
    out_ref[...] = out + bp[...]


def _tc_mlp(mu, mi, gmf, W1, b1, W2, b2, W3, b3, Wp, bp):
    full = lambda shape: pl.BlockSpec(shape, lambda i: (0, 0))
    grid = (_B // _BLK,)
    return pl.pallas_call(
        _mlp_body,
        grid=grid,
        in_specs=[
            pl.BlockSpec((_BLK, _FM), lambda i: (i, 0)),
            pl.BlockSpec((_BLK, _FM), lambda i: (i, 0)),
            pl.BlockSpec((_BLK, _FG), lambda i: (i, 0)),
            full((128, 128)), full((128, 128)), full((1, 128)),
            full((128, 64)), full((1, 64)),
            full((64, 32)), full((1, 32)),
            full((32, 1)), full((32, 1)), full((1, 1)),
        ],
        out_specs=pl.BlockSpec((_BLK, 1), lambda i: (i, 0)),
        out_shape=jax.ShapeDtypeStruct((_B, 1), jnp.float32),
    )(mu, mi, gmf,
      W1[:128], W1[128:], b1.reshape(1, -1),
      W2, b2.reshape(1, -1), W3, b3.reshape(1, -1),
      Wp[:32], Wp[32:], bp.reshape(1, 1))


def kernel(user, item, embed_user_GMF, embed_item_GMF, embed_user_MLP,
           embed_item_MLP, W1, b1, W2, b2, W3, b3, Wp, bp):
    user = user.astype(jnp.int32)
    item = item.astype(jnp.int32)
    mu, mi = _sc_mlp_gather(user, item, embed_user_MLP, embed_item_MLP)
    pug, pig = _pack_tables(embed_user_GMF, embed_item_GMF)
    gmf = _sc_gmf(user.reshape(_NW, _NCH, _CH), item.reshape(_NW, _NCH, _CH),
                  pug, pig)
    out = _tc_mlp(mu, mi, gmf, W1, b1, W2, b2, W3, b3, Wp, bp)
    return out.reshape(-1)


# speedup vs baseline: 1.5541x; 1.0264x over previous
import functools

import jax
import jax.numpy as jnp
from jax import lax
from jax.experimental import pallas as pl
from jax.experimental.pallas import tpu as pltpu
from jax.experimental.pallas import tpu_sc as plsc

_B = 16384
_U = 1000000
_CHU = 16384
_GRID_PACK = (_U + _CHU - 1) // _CHU
_RP = _GRID_PACK * (_CHU // 4)
_UPAD = _RP * 4
_NC = 2
_NS = 16
_NW = _NC * _NS
_BPW = _B // _NW
_CH = 128
_NCH = _BPW // _CH
_FG = 32
_FM = 128


def _pack_body(xu_ref, xi_ref, ou_ref, oi_ref):
    q_span = _CHU // 4
    eye = jnp.eye(128, dtype=jnp.float32)
    j = pl.program_id(0)
    valid = _U - j * _CHU
    col = lax.broadcasted_iota(jnp.int32, (32, _CHU), 1)
    in_bounds = col < valid
    for x_ref, o_ref in ((xu_ref, ou_ref), (xi_ref, oi_ref)):
        x = jnp.where(in_bounds, x_ref[...], 0.0)
        xc = jnp.concatenate([x[:, q * q_span:(q + 1) * q_span]
                              for q in range(4)], axis=0)
        o_ref[...] = jax.lax.dot_general(
            xc, eye, (((0,), (0,)), ((), ())),
            preferred_element_type=jnp.float32)


def _pack_tables(eug, eig):
    pu, pi = pl.pallas_call(
        _pack_body,
        grid=(_GRID_PACK,),
        in_specs=[pl.BlockSpec((32, _CHU), lambda j: (0, j)),
                  pl.BlockSpec((32, _CHU), lambda j: (0, j))],
        out_specs=[pl.BlockSpec((_CHU // 4, 128), lambda j: (j, 0)),
                   pl.BlockSpec((_CHU // 4, 128), lambda j: (j, 0))],
        out_shape=[jax.ShapeDtypeStruct((_RP, 128), jnp.float32),
                   jax.ShapeDtypeStruct((_RP, 128), jnp.float32)],
    )(eug.T, eig.T)
    return pu.reshape(_UPAD, 32), pi.reshape(_UPAD, 32)


def _mesh():
    return plsc.VectorSubcoreMesh(core_axis_name="c", subcore_axis_name="s",
                                  num_cores=_NC, num_subcores=_NS)


def _wid():
    return lax.axis_index("s") * _NC + lax.axis_index("c")


def _gmf_body(user_hbm, item_hbm, eug, eig, gmf_out,
              idx_u, idx_i, vid_u, vid_i, gu_v, gi_v, sem):
    wid = _wid()
    pltpu.sync_copy(user_hbm.at[wid], idx_u)
    pltpu.sync_copy(item_hbm.at[wid], idx_i)

    sb = _CHU.bit_length() - 1
    ss = sb - 2
    sm = (1 << ss) - 1

    def remap(k, carry):
        for src, dst in ((idx_u, vid_u), (idx_i, vid_i)):
            for c in range(_NCH):
                v = src[c, pl.ds(k * 16, 16)]
                vrow = (jax.lax.shift_left(jax.lax.shift_right_logical(v, sb),
                                           sb)
                        + jax.lax.shift_left(v & sm, 2)
                        + (jax.lax.shift_right_logical(v, ss) & 3))
                dst[c, pl.ds(k * 16, 16)] = vrow
        return carry

    lax.fori_loop(0, _CH // 16, remap, 0)
    descs = []
    for c in range(_NCH):
        descs.append(pltpu.async_copy(eug.at[vid_u.at[c]], gu_v.at[c], sem))
        descs.append(pltpu.async_copy(eig.at[vid_i.at[c]], gi_v.at[c], sem))
    for d in descs:
        d.wait()

    def mul_row(r, carry):
        for c in range(_NCH):
            for j in range(_FG // 16):
                sl = pl.ds(j * 16, 16)
                gu_v[c, r, sl] = gu_v[c, r, sl] * gi_v[c, r, sl]
        return carry

    lax.fori_loop(0, _CH, mul_row, 0)
    pltpu.sync_copy(gu_v, gmf_out.at[wid])


def _sc_gmf(user3, item3, eug, eig):
    k = functools.partial(
        pl.kernel, mesh=_mesh(),
        compiler_params=pltpu.CompilerParams(use_tc_tiling_on_sc=False),
        out_type=jax.ShapeDtypeStruct((_NW, _NCH, _CH, _FG), jnp.float32),
        scratch_types=[
            pltpu.VMEM((_NCH, _CH), jnp.int32),
            pltpu.VMEM((_NCH, _CH), jnp.int32),
            pltpu.VMEM((_NCH, _CH), jnp.int32),
            pltpu.VMEM((_NCH, _CH), jnp.int32),
            pltpu.VMEM((_NCH, _CH, _FG), jnp.float32),
            pltpu.VMEM((_NCH, _CH, _FG), jnp.float32),
            pltpu.SemaphoreType.DMA,
        ],
    )(_gmf_body)
    return k(user3, item3, eug, eig).reshape(_B, _FG)


def _mlp_gather_body(user_hbm, item_hbm, eum, eim, mu_out, mi_out,
                     idx_u, idx_i, bufs_u, bufs_i, sem_u, sem_i):
    wid = _wid()
    base = wid * _BPW
    for c in range(_NCH):
        pltpu.sync_copy(user_hbm.at[pl.ds(base + c * _CH, _CH)], idx_u.at[c])
        pltpu.sync_copy(item_hbm.at[pl.ds(base + c * _CH, _CH)], idx_i.at[c])
    du = [None, None]
    di = [None, None]
    du[0] = pltpu.async_copy(eum.at[idx_u.at[0]], bufs_u.at[0], sem_u.at[0])
    di[0] = pltpu.async_copy(eim.at[idx_i.at[0]], bufs_i.at[0], sem_i.at[0])
    for c in range(_NCH):
        s = c % 2
        if c + 1 < _NCH:
            n = (c + 1) % 2
            du[n] = pltpu.async_copy(eum.at[idx_u.at[c + 1]], bufs_u.at[n],
                                     sem_u.at[n])
            di[n] = pltpu.async_copy(eim.at[idx_i.at[c + 1]], bufs_i.at[n],
                                     sem_i.at[n])
        du[s].wait()
        pltpu.sync_copy(bufs_u.at[s], mu_out.at[pl.ds(base + c * _CH, _CH)])
        di[s].wait()
        pltpu.sync_copy(bufs_i.at[s], mi_out.at[pl.ds(base + c * _CH, _CH)])


def _sc_mlp_gather(user, item, eum, eim):
    k = functools.partial(
        pl.kernel, mesh=_mesh(),
        out_type=(
            jax.ShapeDtypeStruct((_B, _FM), jnp.float32),
            jax.ShapeDtypeStruct((_B, _FM), jnp.float32),
        ),
        scratch_types=[
            pltpu.VMEM((_NCH, _CH), jnp.int32),
            pltpu.VMEM((_NCH, _CH), jnp.int32),
            pltpu.VMEM((2, _CH, _FM), jnp.float32),
            pltpu.VMEM((2, _CH, _FM), jnp.float32),
            pltpu.SemaphoreType.DMA((2,)),
            pltpu.SemaphoreType.DMA((2,)),
        ],
    )(_mlp_gather_body)
    return k(user, item, eum, eim)


_BLK = 2048


def _mlp_body(mu_ref, mi_ref, gmf_ref, w1a, w1b, b1, w2, b2, w3, b3,
              wpa, wpb, bp, out_ref):
    x = jnp.dot(mu_ref[...], w1a[...], preferred_element_type=jnp.float32)
    x = x + jnp.dot(mi_ref[...], w1b[...], preferred_element_type=jnp.float32)
    x = jnp.maximum(x + b1[...], 0.0)
    x = jnp.maximum(jnp.dot(x, w2[...], preferred_element_type=jnp.float32)
                    + b2[...], 0.0)
    x = jnp.maximum(jnp.dot(x, w3[...], preferred_element_type=jnp.float32)
                    + b3[...], 0.0)
    out = jax.lax.dot_general(wpa[...], gmf_ref[...], (((0,), (1,)), ((), ())),
                              preferred_element_type=jnp.float32)
    out = out + jax.lax.dot_general(wpb[...], x, (((0,), (1,)), ((), ())),
                                    preferred_element_type=jnp.float32)
    out_ref[pl.ds(pl.program_id(0), 1), :] = out + bp[...]


def _tc_mlp(mu, mi, gmf, W1, b1, W2, b2, W3, b3, Wp, bp):
    full = lambda shape: pl.BlockSpec(shape, lambda i: (0, 0))
    grid = (_B // _BLK,)
    return pl.pallas_call(
        _mlp_body,
        grid=grid,
        in_specs=[
            pl.BlockSpec((_BLK, _FM), lambda i: (i, 0)),
            pl.BlockSpec((_BLK, _FM), lambda i: (i, 0)),
            pl.BlockSpec((_BLK, _FG), lambda i: (i, 0)),
            full((128, 128)), full((128, 128)), full((1, 128)),
            full((128, 64)), full((1, 64)),
            full((64, 32)), full((1, 32)),
            full((32, 1)), full((32, 1)), full((1, 1)),
        ],
        out_specs=pl.BlockSpec((_B // _BLK, _BLK), lambda i: (0, 0)),
        out_shape=jax.ShapeDtypeStruct((_B // _BLK, _BLK), jnp.float32),
    )(mu, mi, gmf,
      W1[:128], W1[128:], b1.reshape(1, -1),
      W2, b2.reshape(1, -1), W3, b3.reshape(1, -1),
      Wp[:32], Wp[32:], bp.reshape(1, 1))


def kernel(user, item, embed_user_GMF, embed_item_GMF, embed_user_MLP,
           embed_item_MLP, W1, b1, W2, b2, W3, b3, Wp, bp):
    user = user.astype(jnp.int32)
    item = item.astype(jnp.int32)
    mu, mi = _sc_mlp_gather(user, item, embed_user_MLP, embed_item_MLP)
    pug, pig = _pack_tables(embed_user_GMF, embed_item_GMF)
    gmf = _sc_gmf(user.reshape(_NW, _NCH, _CH), item.reshape(_NW, _NCH, _CH),
                  pug, pig)
    out = _tc_mlp(mu, mi, gmf, W1, b1, W2, b2, W3, b3, Wp, bp)
    return out.reshape(-1)
